# Initial kernel scaffold; baseline (speedup 1.0000x reference)
import jax
import jax.numpy as jnp
from jax.experimental import pallas as pl


def kernel(features, coords, segment_ids):
    raise NotImplementedError("write your pallas kernel here")


# speedup vs baseline: 3.6623x; 3.6623x over previous
import functools

import jax
import jax.numpy as jnp
from jax import lax
from jax.experimental import pallas as pl
from jax.experimental.pallas import tpu as pltpu
from jax.experimental.pallas import tpu_sc as plsc

C, H, W = 16, 1024, 1024
N = H * W
S = 2048

_NC = 2
_NS = 16
_NW = _NC * _NS
_PPW = N // _NW
_K = 128
_NCHUNK = _PPW // _K


_TR_R = 8


def _tr_body(x_ref, o_ref):
    x = x_ref[...]
    o_ref[...] = x.reshape(C, _TR_R * W).T


_transpose = pl.pallas_call(
    _tr_body,
    grid=(H // _TR_R,),
    in_specs=[pl.BlockSpec((C, _TR_R, W), lambda i: (0, i, 0))],
    out_specs=pl.BlockSpec((_TR_R * W, C), lambda i: (i, 0)),
    out_shape=jax.ShapeDtypeStruct((N, C), jnp.float32),
)


_mesh = plsc.VectorSubcoreMesh(core_axis_name="c", subcore_axis_name="s")


@functools.partial(
    pl.kernel,
    out_type=(
        jax.ShapeDtypeStruct((_NC, S, C), jnp.float32),
        jax.ShapeDtypeStruct((_NC, S), jnp.float32),
    ),
    mesh=_mesh,
    compiler_params=pltpu.CompilerParams(use_tc_tiling_on_sc=False),
    scratch_types=[
        pltpu.VMEM((_K,), jnp.int32),
        pltpu.VMEM((_K,), jnp.int32),
        pltpu.VMEM((_K,), jnp.int32),
        pltpu.VMEM((_K,), jnp.int32),
        pltpu.VMEM((_K, C), jnp.float32),
        pltpu.VMEM((_K,), jnp.float32),
        pltpu.VMEM_SHARED((S, C), jnp.float32),
        pltpu.VMEM_SHARED((S,), jnp.float32),
        pltpu.SemaphoreType.DMA,
    ],
)
def _sc_segsum(feat_t, rr_hbm, cc_hbm, seg_hbm, zsum, zcnt,
               sums_out, cnts_out,
               rr_v, cc_v, seg_v, idx_v, rows_v, ones_v, acc_s, cnt_s, sem):
    ci = lax.axis_index("c")
    si = lax.axis_index("s")
    wid = si * _NC + ci

    @pl.when(si == 0)
    def _():
        pltpu.sync_copy(zsum, acc_s)
        pltpu.sync_copy(zcnt, cnt_s)

    for j in range(_K // 16):
        ones_v[pl.ds(j * 16, 16)] = jnp.ones((16,), jnp.float32)

    plsc.subcore_barrier()

    def chunk_body(t, carry):
        base = wid * _PPW + t * _K
        pltpu.sync_copy(rr_hbm.at[pl.ds(base, _K)], rr_v)
        pltpu.sync_copy(cc_hbm.at[pl.ds(base, _K)], cc_v)
        pltpu.sync_copy(seg_hbm.at[pl.ds(base, _K)], seg_v)
        for j in range(_K // 16):
            sl = pl.ds(j * 16, 16)
            idx_v[sl] = rr_v[sl] * W + cc_v[sl]
        pltpu.async_copy(feat_t.at[idx_v], rows_v, sem).wait()
        pltpu.sync_copy(rows_v, acc_s.at[seg_v], add=True)
        pltpu.sync_copy(ones_v, cnt_s.at[seg_v], add=True)
        return carry

    lax.fori_loop(0, _NCHUNK, chunk_body, 0)

    plsc.subcore_barrier()

    @pl.when(si == 0)
    def _():
        pltpu.sync_copy(acc_s, sums_out.at[ci])
        pltpu.sync_copy(cnt_s, cnts_out.at[ci])


def _merge_body(s_ref, c_ref, o_ref):
    ssum = s_ref[0] + s_ref[1]
    cnt = c_ref[0] + c_ref[1]
    o_ref[...] = ssum / jnp.maximum(cnt, 1.0)[:, None]


_merge = pl.pallas_call(
    _merge_body,
    out_shape=jax.ShapeDtypeStruct((S, C), jnp.float32),
)


def kernel(features, coords, segment_ids):
    feat_t = _transpose(features)
    rr = coords[:, 0]
    cc = coords[:, 1]
    zsum = jnp.zeros((S, C), jnp.float32)
    zcnt = jnp.zeros((S,), jnp.float32)
    sums, cnts = _sc_segsum(feat_t, rr, cc, segment_ids, zsum, zcnt)
    return _merge(sums, cnts)


# speedup vs baseline: 5.0685x; 1.3840x over previous
import functools

import jax
import jax.numpy as jnp
from jax import lax
from jax.experimental import pallas as pl
from jax.experimental.pallas import tpu as pltpu
from jax.experimental.pallas import tpu_sc as plsc

C, H, W = 16, 1024, 1024
N = H * W
S = 2048

_NC = 2
_NS = 16
_NW = _NC * _NS
_PPW = N // _NW
_K = 128
_NCHUNK = _PPW // _K

_TK = 2048
_TCHUNK = _PPW // _TK

_mesh = plsc.VectorSubcoreMesh(core_axis_name="c", subcore_axis_name="s")
_sc_params = pltpu.CompilerParams(use_tc_tiling_on_sc=False, needs_layout_passes=False)


@functools.partial(
    pl.kernel,
    out_type=jax.ShapeDtypeStruct((N * C,), jnp.float32),
    mesh=_mesh,
    compiler_params=_sc_params,
    scratch_types=[
        pltpu.VMEM((C * _TK,), jnp.float32),
        pltpu.VMEM((_TK * C,), jnp.float32),
        pltpu.SemaphoreType.DMA,
    ],
)
def _sc_transpose(featc, table, ch_v, rows_v, sem):
    ci = lax.axis_index("c")
    si = lax.axis_index("s")
    wid = si * _NC + ci
    base = wid * _PPW
    iota16 = lax.iota(jnp.int32, 16)

    def chunk_body(t, carry):
        cbase = base + t * _TK
        copies = [
            pltpu.async_copy(
                featc.at[pl.ds(c * N + cbase, _TK)],
                ch_v.at[pl.ds(c * _TK, _TK)],
                sem,
            )
            for c in range(C)
        ]
        for cp in copies:
            cp.wait()

        def group_body(g, carry2):
            flat_base = g * (16 * C) + iota16 * C
            for c in range(C):
                val = ch_v[pl.ds(c * _TK + g * 16, 16)]
                plsc.store_scatter(rows_v, [flat_base + c], val)
            return carry2

        lax.fori_loop(0, _TK // 16, group_body, 0)
        pltpu.sync_copy(rows_v, table.at[pl.ds(cbase * C, _TK * C)])
        return carry

    lax.fori_loop(0, _TCHUNK, chunk_body, 0)


@functools.partial(
    pl.kernel,
    out_type=(
        jax.ShapeDtypeStruct((_NC, S, C), jnp.float32),
        jax.ShapeDtypeStruct((_NC, S), jnp.float32),
    ),
    mesh=_mesh,
    compiler_params=_sc_params,
    scratch_types=[
        pltpu.VMEM((_K,), jnp.int32),
        pltpu.VMEM((_K,), jnp.int32),
        pltpu.VMEM((_K,), jnp.int32),
        pltpu.VMEM((_K,), jnp.int32),
        pltpu.VMEM((_K, C), jnp.float32),
        pltpu.VMEM((_K,), jnp.float32),
        pltpu.VMEM_SHARED((S, C), jnp.float32),
        pltpu.VMEM_SHARED((S,), jnp.float32),
        pltpu.SemaphoreType.DMA,
    ],
)
def _sc_segsum(feat_t, rr_hbm, cc_hbm, seg_hbm, zsum, zcnt,
               sums_out, cnts_out,
               rr_v, cc_v, seg_v, idx_v, rows_v, ones_v, acc_s, cnt_s, sem):
    ci = lax.axis_index("c")
    si = lax.axis_index("s")
    wid = si * _NC + ci

    @pl.when(si == 0)
    def _():
        pltpu.sync_copy(zsum, acc_s)
        pltpu.sync_copy(zcnt, cnt_s)

    for j in range(_K // 16):
        ones_v[pl.ds(j * 16, 16)] = jnp.ones((16,), jnp.float32)

    plsc.subcore_barrier()

    def chunk_body(t, carry):
        base = wid * _PPW + t * _K
        pltpu.sync_copy(rr_hbm.at[pl.ds(base, _K)], rr_v)
        pltpu.sync_copy(cc_hbm.at[pl.ds(base, _K)], cc_v)
        pltpu.sync_copy(seg_hbm.at[pl.ds(base, _K)], seg_v)
        for j in range(_K // 16):
            sl = pl.ds(j * 16, 16)
            idx_v[sl] = rr_v[sl] * W + cc_v[sl]
        pltpu.async_copy(feat_t.at[idx_v], rows_v, sem).wait()
        pltpu.sync_copy(rows_v, acc_s.at[seg_v], add=True)
        pltpu.sync_copy(ones_v, cnt_s.at[seg_v], add=True)
        return carry

    lax.fori_loop(0, _NCHUNK, chunk_body, 0)

    plsc.subcore_barrier()

    @pl.when(si == 0)
    def _():
        pltpu.sync_copy(acc_s, sums_out.at[ci])
        pltpu.sync_copy(cnt_s, cnts_out.at[ci])


def _merge_body(s_ref, c_ref, o_ref):
    ssum = s_ref[0] + s_ref[1]
    cnt = c_ref[0] + c_ref[1]
    o_ref[...] = ssum / jnp.maximum(cnt, 1.0)[:, None]


_merge = pl.pallas_call(
    _merge_body,
    out_shape=jax.ShapeDtypeStruct((S, C), jnp.float32),
)


def kernel(features, coords, segment_ids):
    featc = features.reshape(C * N)
    feat_t = _sc_transpose(featc).reshape(N, C)
    rr = coords[:, 0]
    cc = coords[:, 1]
    zsum = jnp.zeros((S, C), jnp.float32)
    zcnt = jnp.zeros((S,), jnp.float32)
    sums, cnts = _sc_segsum(feat_t, rr, cc, segment_ids, zsum, zcnt)
    return _merge(sums, cnts)


# speedup vs baseline: 11.2973x; 2.2289x over previous
import functools

import jax
import jax.numpy as jnp
from jax import lax
from jax.experimental import pallas as pl
from jax.experimental.pallas import tpu as pltpu
from jax.experimental.pallas import tpu_sc as plsc

C, H, W = 16, 1024, 1024
N = H * W
S = 2048

_NC = 2
_NS = 16
_NW = _NC * _NS
_PPW = N // _NW

_TK = 2048
_TCHUNK = _PPW // _TK

_B = 1024
_NBLK = _PPW // _B
_GPB = _B // 16

_mesh = plsc.VectorSubcoreMesh(core_axis_name="c", subcore_axis_name="s")
_sc_params = pltpu.CompilerParams(
    use_tc_tiling_on_sc=False, needs_layout_passes=False
)


@functools.partial(
    pl.kernel,
    out_type=jax.ShapeDtypeStruct((N, C), jnp.float32),
    mesh=_mesh,
    compiler_params=_sc_params,
    scratch_types=[
        pltpu.VMEM((C * _TK,), jnp.float32),
        pltpu.VMEM((_TK, C), jnp.float32),
        pltpu.SemaphoreType.DMA,
    ],
)
def _sc_transpose(featc, table, ch_v, rows_v, sem):
    ci = lax.axis_index("c")
    si = lax.axis_index("s")
    wid = si * _NC + ci
    base = wid * _PPW
    iota16 = lax.iota(jnp.int32, 16)

    def chunk_body(t, carry):
        cbase = base + t * _TK
        copies = [
            pltpu.async_copy(
                featc.at[pl.ds(c * N + cbase, _TK)],
                ch_v.at[pl.ds(c * _TK, _TK)],
                sem,
            )
            for c in range(C)
        ]
        for cp in copies:
            cp.wait()

        def group_body(g, carry2):
            rows_idx = g * 16 + iota16
            for c in range(C):
                val = ch_v[pl.ds(c * _TK + g * 16, 16)]
                col_idx = jnp.full((16,), c, jnp.int32)
                plsc.store_scatter(rows_v, [rows_idx, col_idx], val)
            return carry2

        lax.fori_loop(0, _TK // 16, group_body, 0)
        pltpu.sync_copy(rows_v, table.at[pl.ds(cbase, _TK)])
        return carry

    lax.fori_loop(0, _TCHUNK, chunk_body, 0)


@functools.partial(
    pl.kernel,
    out_type=(
        jax.ShapeDtypeStruct((_NW, S * C), jnp.float32),
        jax.ShapeDtypeStruct((_NW, S * C), jnp.float32),
    ),
    mesh=_mesh,
    compiler_params=_sc_params,
    scratch_types=[
        pltpu.VMEM((2, _B), jnp.int32),
        pltpu.VMEM((2, _B), jnp.int32),
        pltpu.VMEM((2, _B), jnp.int32),
        pltpu.VMEM((2, _B), jnp.int32),
        pltpu.VMEM((2, _B, C), jnp.float32),
        pltpu.VMEM((S * C,), jnp.float32),
        pltpu.VMEM((S * C,), jnp.float32),
        pltpu.SemaphoreType.DMA,
        pltpu.SemaphoreType.DMA,
        pltpu.SemaphoreType.DMA,
        pltpu.SemaphoreType.DMA,
    ],
)
def _sc_segsum(feat_t, rr_hbm, cc_hbm, seg_hbm,
               sums_out, cnts_out,
               rr_v, cc_v, seg_v, idx_v, rows_v, acc_f, cnt_f,
               semi0, semi1, semg0, semg1):
    ci = lax.axis_index("c")
    si = lax.axis_index("s")
    wid = si * _NC + ci
    tbase = wid * _PPW
    semi = (semi0, semi1)
    semg = (semg0, semg1)
    zero16 = jnp.zeros((16,), jnp.float32)

    def zbody(i, carry):
        acc_f[pl.ds(i * 16, 16)] = zero16
        cnt_f[pl.ds(i * 16, 16)] = zero16
        return carry

    lax.fori_loop(0, S * C // 16, zbody, 0)

    def fire_inputs(bb, blk):
        gb = tbase + blk * _B
        pltpu.async_copy(rr_hbm.at[pl.ds(gb, _B)], rr_v.at[bb], semi[bb])
        pltpu.async_copy(cc_hbm.at[pl.ds(gb, _B)], cc_v.at[bb], semi[bb])
        pltpu.async_copy(seg_hbm.at[pl.ds(gb, _B)], seg_v.at[bb], semi[bb])

    def wait_inputs(bb):
        pltpu.make_async_copy(rr_hbm.at[pl.ds(0, _B)], rr_v.at[bb], semi[bb]).wait()
        pltpu.make_async_copy(cc_hbm.at[pl.ds(0, _B)], cc_v.at[bb], semi[bb]).wait()
        pltpu.make_async_copy(seg_hbm.at[pl.ds(0, _B)], seg_v.at[bb], semi[bb]).wait()

    def compute_idx(bb):
        def ibody(g, carry):
            sl = pl.ds(g * 16, 16)
            idx_v[bb, sl] = rr_v[bb, sl] * W + cc_v[bb, sl]
            return carry

        lax.fori_loop(0, _GPB, ibody, 0)

    def fire_gathers(bb):
        for j in range(_B // 128):
            pltpu.async_copy(
                feat_t.at[idx_v.at[bb, pl.ds(j * 128, 128)]],
                rows_v.at[bb, pl.ds(j * 128, 128)],
                semg[bb],
            )

    def wait_gathers(bb):
        pltpu.make_async_copy(
            feat_t.at[pl.ds(0, _B)], rows_v.at[bb], semg[bb]
        ).wait()

    def flush(p, a, n):
        sl = pl.ds(p * C, C)
        acc_f[sl] = acc_f[sl] + a
        cnt_f[sl] = cnt_f[sl] + n

    def process(bb, carry):
        def group(g, carry2):
            s = seg_v[bb, pl.ds(g * 16, 16)]
            s0 = s[0]
            s15 = s[15]
            base_px = g * 16

            def uniform_fn(c3):
                rs = rows_v[bb, base_px]
                for l in range(1, 16):
                    rs = rs + rows_v[bb, base_px + l]

                def same_fn(c4):
                    a, n, p = c4
                    return (a + rs, n + 1.0, p)

                def diff_fn(c4):
                    a, n, p = c4
                    flush(p, a, n)
                    return (rs, jnp.ones((16,), jnp.float32), s0)

                return lax.cond(s0 == c3[2], same_fn, diff_fn, c3)

            def mixed_fn(c3):
                c4 = c3
                for l in range(16):
                    sl_ = s[l]
                    row = rows_v[bb, base_px + l]

                    def same_fn(c5, row=row):
                        a, n, p = c5
                        return (a + row, n + 0.0625, p)

                    def diff_fn(c5, row=row, sl_=sl_):
                        a, n, p = c5
                        flush(p, a, n)
                        return (row, jnp.full((16,), 0.0625, jnp.float32), sl_)

                    c4 = lax.cond(sl_ == c4[2], same_fn, diff_fn, c4)
                return c4

            return lax.cond(s0 == s15, uniform_fn, mixed_fn, carry2)

        return lax.fori_loop(0, _GPB, group, carry)

    fire_inputs(0, 0)
    wait_inputs(0)
    compute_idx(0)
    fire_gathers(0)
    fire_inputs(1, 1)
    sfirst = seg_v[0, pl.ds(0, 16)]
    carry = (zero16, zero16, sfirst[0])

    def block_step(t, carry):
        for b in (0, 1):
            blk = 2 * t + b
            nb = 1 - b

            @pl.when(blk + 1 < _NBLK)
            def _():
                wait_inputs(nb)
                compute_idx(nb)
                fire_gathers(nb)

            wait_gathers(b)
            carry = process(b, carry)

            @pl.when(blk + 2 < _NBLK)
            def _():
                fire_inputs(b, blk + 2)
        return carry

    carry = lax.fori_loop(0, _NBLK // 2, block_step, carry)
    a, n, p = carry
    flush(p, a, n)

    pltpu.sync_copy(acc_f, sums_out.at[wid])
    pltpu.sync_copy(cnt_f, cnts_out.at[wid])


def _merge_body(s_ref, c_ref, o_ref):
    sm = jnp.sum(s_ref[...], axis=0).reshape(S * C // 128, 128)
    cn = jnp.sum(c_ref[...], axis=0).reshape(S * C // 128, 128)
    ii = lax.broadcasted_iota(jnp.int32, (128, 128), 0)
    jj = lax.broadcasted_iota(jnp.int32, (128, 128), 1)
    m = ((ii // C) == (jj // C)).astype(jnp.float32)
    nb = jax.lax.dot(cn, m, precision=jax.lax.Precision.HIGHEST)
    o_ref[...] = sm / jnp.maximum(nb, 1.0)


_merge = pl.pallas_call(
    _merge_body,
    out_shape=jax.ShapeDtypeStruct((S * C // 128, 128), jnp.float32),
)


def kernel(features, coords, segment_ids):
    featc = features.reshape(C * N)
    feat_t = _sc_transpose(featc)
    rr = coords[:, 0]
    cc = coords[:, 1]
    sums, cnts = _sc_segsum(feat_t, rr, cc, segment_ids)
    return _merge(sums, cnts).reshape(S, C)


# speedup vs baseline: 12.9107x; 1.1428x over previous
import functools

import jax
import jax.numpy as jnp
from jax import lax
from jax.experimental import pallas as pl
from jax.experimental.pallas import tpu as pltpu
from jax.experimental.pallas import tpu_sc as plsc

C, H, W = 16, 1024, 1024
N = H * W
S = 2048

_NC = 2
_NS = 16
_NW = _NC * _NS
_PPW = N // _NW

_TK = 1024
_TCHUNK = _PPW // _TK

_B = 1024
_NBLK = _PPW // _B
_GPB = _B // 16

_mesh = plsc.VectorSubcoreMesh(core_axis_name="c", subcore_axis_name="s")
_sc_params = pltpu.CompilerParams(
    use_tc_tiling_on_sc=False, needs_layout_passes=False
)


@functools.partial(
    pl.kernel,
    out_type=jax.ShapeDtypeStruct((N, C), jnp.float32),
    mesh=_mesh,
    compiler_params=_sc_params,
    scratch_types=[
        pltpu.VMEM((2, C * _TK), jnp.float32),
        pltpu.VMEM((2, _TK, C), jnp.float32),
        pltpu.SemaphoreType.DMA,
        pltpu.SemaphoreType.DMA,
        pltpu.SemaphoreType.DMA,
        pltpu.SemaphoreType.DMA,
    ],
)
def _sc_transpose(featc, table, ch_v, rows_v, semc0, semc1, semo0, semo1):
    ci = lax.axis_index("c")
    si = lax.axis_index("s")
    wid = si * _NC + ci
    base = wid * _PPW
    iota16 = lax.iota(jnp.int32, 16)
    cvecs = [jnp.full((16,), c, jnp.int32) for c in range(C)]
    semc = (semc0, semc1)
    semo = (semo0, semo1)

    def fire_in(bb, t):
        cbase = base + t * _TK
        for c in range(C):
            pltpu.async_copy(
                featc.at[pl.ds(c * N + cbase, _TK)],
                ch_v.at[bb, pl.ds(c * _TK, _TK)],
                semc[bb],
            )

    def wait_in(bb):
        pltpu.make_async_copy(
            featc.at[pl.ds(0, C * _TK)], ch_v.at[bb], semc[bb]
        ).wait()

    def process(bb):
        def group_body(g, carry2):
            win = rows_v.at[bb, pl.ds(g * 16, 16)]
            for c in range(C):
                val = ch_v[bb, pl.ds(c * _TK + g * 16, 16)]
                plsc.store_scatter(win, [iota16, cvecs[c]], val)
            return carry2

        lax.fori_loop(0, _TK // 16, group_body, 0)

    fire_in(0, 0)
    fire_in(1, 1)

    def chunk_step(tt, carry):
        for b in (0, 1):
            t = 2 * tt + b
            wait_in(b)

            @pl.when(t >= 2)
            def _():
                pltpu.make_async_copy(
                    rows_v.at[b], table.at[pl.ds(0, _TK)], semo[b]
                ).wait()

            process(b)

            @pl.when(t + 2 < _TCHUNK)
            def _():
                fire_in(b, t + 2)

            pltpu.async_copy(
                rows_v.at[b],
                table.at[pl.ds(base + t * _TK, _TK)],
                semo[b],
            )
        return carry

    lax.fori_loop(0, _TCHUNK // 2, chunk_step, 0)
    pltpu.make_async_copy(rows_v.at[0], table.at[pl.ds(0, _TK)], semo[0]).wait()
    pltpu.make_async_copy(rows_v.at[1], table.at[pl.ds(0, _TK)], semo[1]).wait()


@functools.partial(
    pl.kernel,
    out_type=(
        jax.ShapeDtypeStruct((_NW, S * C), jnp.float32),
        jax.ShapeDtypeStruct((_NW, S * C), jnp.float32),
    ),
    mesh=_mesh,
    compiler_params=_sc_params,
    scratch_types=[
        pltpu.VMEM((2, _B), jnp.int32),
        pltpu.VMEM((2, _B), jnp.int32),
        pltpu.VMEM((2, _B), jnp.int32),
        pltpu.VMEM((2, _B), jnp.int32),
        pltpu.VMEM((2, _B, C), jnp.float32),
        pltpu.VMEM((S * C,), jnp.float32),
        pltpu.VMEM((S * C,), jnp.float32),
        pltpu.SemaphoreType.DMA,
        pltpu.SemaphoreType.DMA,
        pltpu.SemaphoreType.DMA,
        pltpu.SemaphoreType.DMA,
    ],
)
def _sc_segsum(feat_t, rr_hbm, cc_hbm, seg_hbm,
               sums_out, cnts_out,
               rr_v, cc_v, seg_v, idx_v, rows_v, acc_f, cnt_f,
               semi0, semi1, semg0, semg1):
    ci = lax.axis_index("c")
    si = lax.axis_index("s")
    wid = si * _NC + ci
    tbase = wid * _PPW
    semi = (semi0, semi1)
    semg = (semg0, semg1)
    zero16 = jnp.zeros((16,), jnp.float32)

    def zbody(i, carry):
        acc_f[pl.ds(i * 16, 16)] = zero16
        cnt_f[pl.ds(i * 16, 16)] = zero16
        return carry

    lax.fori_loop(0, S * C // 16, zbody, 0)

    def fire_inputs(bb, blk):
        gb = tbase + blk * _B
        pltpu.async_copy(rr_hbm.at[pl.ds(gb, _B)], rr_v.at[bb], semi[bb])
        pltpu.async_copy(cc_hbm.at[pl.ds(gb, _B)], cc_v.at[bb], semi[bb])
        pltpu.async_copy(seg_hbm.at[pl.ds(gb, _B)], seg_v.at[bb], semi[bb])

    def wait_inputs(bb):
        pltpu.make_async_copy(rr_hbm.at[pl.ds(0, _B)], rr_v.at[bb], semi[bb]).wait()
        pltpu.make_async_copy(cc_hbm.at[pl.ds(0, _B)], cc_v.at[bb], semi[bb]).wait()
        pltpu.make_async_copy(seg_hbm.at[pl.ds(0, _B)], seg_v.at[bb], semi[bb]).wait()

    def compute_idx(bb):
        def ibody(g, carry):
            sl = pl.ds(g * 16, 16)
            idx_v[bb, sl] = rr_v[bb, sl] * W + cc_v[bb, sl]
            return carry

        lax.fori_loop(0, _GPB, ibody, 0)

    def fire_gathers(bb):
        for j in range(_B // 128):
            pltpu.async_copy(
                feat_t.at[idx_v.at[bb, pl.ds(j * 128, 128)]],
                rows_v.at[bb, pl.ds(j * 128, 128)],
                semg[bb],
            )

    def wait_gathers(bb):
        pltpu.make_async_copy(
            feat_t.at[pl.ds(0, _B)], rows_v.at[bb], semg[bb]
        ).wait()

    def flush(p, a, n):
        sl = pl.ds(p * C, C)
        acc_f[sl] = acc_f[sl] + a
        cnt_f[sl] = cnt_f[sl] + n

    def process(bb, carry):
        def group(g, carry2):
            s = seg_v[bb, pl.ds(g * 16, 16)]
            s0 = s[0]
            s15 = s[15]
            base_px = g * 16

            def uniform_fn(c3):
                rs = rows_v[bb, base_px]
                for l in range(1, 16):
                    rs = rs + rows_v[bb, base_px + l]

                def same_fn(c4):
                    a, n, p = c4
                    return (a + rs, n + 1.0, p)

                def diff_fn(c4):
                    a, n, p = c4
                    flush(p, a, n)
                    return (rs, jnp.ones((16,), jnp.float32), s0)

                return lax.cond(s0 == c3[2], same_fn, diff_fn, c3)

            def mixed_fn(c3):
                c4 = c3
                for l in range(16):
                    sl_ = s[l]
                    row = rows_v[bb, base_px + l]

                    def same_fn(c5, row=row):
                        a, n, p = c5
                        return (a + row, n + 0.0625, p)

                    def diff_fn(c5, row=row, sl_=sl_):
                        a, n, p = c5
                        flush(p, a, n)
                        return (row, jnp.full((16,), 0.0625, jnp.float32), sl_)

                    c4 = lax.cond(sl_ == c4[2], same_fn, diff_fn, c4)
                return c4

            return lax.cond(s0 == s15, uniform_fn, mixed_fn, carry2)

        return lax.fori_loop(0, _GPB, group, carry)

    fire_inputs(0, 0)
    wait_inputs(0)
    compute_idx(0)
    fire_gathers(0)
    fire_inputs(1, 1)
    sfirst = seg_v[0, pl.ds(0, 16)]
    carry = (zero16, zero16, sfirst[0])

    def block_step(t, carry):
        for b in (0, 1):
            blk = 2 * t + b
            nb = 1 - b

            @pl.when(blk + 1 < _NBLK)
            def _():
                wait_inputs(nb)
                compute_idx(nb)
                fire_gathers(nb)

            wait_gathers(b)
            carry = process(b, carry)

            @pl.when(blk + 2 < _NBLK)
            def _():
                fire_inputs(b, blk + 2)
        return carry

    carry = lax.fori_loop(0, _NBLK // 2, block_step, carry)
    a, n, p = carry
    flush(p, a, n)

    pltpu.sync_copy(acc_f, sums_out.at[wid])
    pltpu.sync_copy(cnt_f, cnts_out.at[wid])


def _merge_body(s_ref, c_ref, o_ref):
    sm = jnp.sum(s_ref[...], axis=0).reshape(S * C // 128, 128)
    cn = jnp.sum(c_ref[...], axis=0).reshape(S * C // 128, 128)
    ii = lax.broadcasted_iota(jnp.int32, (128, 128), 0)
    jj = lax.broadcasted_iota(jnp.int32, (128, 128), 1)
    m = ((ii // C) == (jj // C)).astype(jnp.float32)
    nb = jax.lax.dot(cn, m, precision=jax.lax.Precision.HIGHEST)
    o_ref[...] = sm / jnp.maximum(nb, 1.0)


_merge = pl.pallas_call(
    _merge_body,
    out_shape=jax.ShapeDtypeStruct((S * C // 128, 128), jnp.float32),
)


def kernel(features, coords, segment_ids):
    featc = features.reshape(C * N)
    feat_t = _sc_transpose(featc)
    rr = coords[:, 0]
    cc = coords[:, 1]
    sums, cnts = _sc_segsum(feat_t, rr, cc, segment_ids)
    return _merge(sums, cnts).reshape(S, C)
